# baseline (speedup 1.0000x reference)
import functools

import jax
import jax.numpy as jnp
from jax import lax
from jax.experimental import pallas as pl
from jax.experimental.pallas import tpu as pltpu
from jax.experimental.pallas import tpu_sc as plsc

N_NODES = 100000
N_EDGES = 1600000
HEADS = 8
OUT_CH = 8

NC = 2
NS = 16
NW = NC * NS
LANES = 16

BLK = 1280
CHUNK = 128
NCHUNK = BLK // CHUNK
GPC = CHUNK // LANES
NBLOCKS = N_EDGES // BLK
BLK_PER, BLK_REM = divmod(NBLOCKS, NW)
ROWS_PER_TILE = N_NODES // NS
NZCOPY = ROWS_PER_TILE // BLK
ZTAIL = ROWS_PER_TILE - NZCOPY * BLK


def _sc_body(x_hbm, src_hbm, dst_hbm, sv_hbm, dv_hbm, part_hbm,
             srcbuf, dstbuf, xsbuf, xdbuf, contrib, svv, dvv,
             acc, sem_g, sem_a, sem_b, sem_s):
    c = lax.axis_index("c")
    s = lax.axis_index("s")
    wid = s * NC + c

    pltpu.sync_copy(sv_hbm, svv)
    pltpu.sync_copy(dv_hbm, dvv)

    def _zero_rows(i, _):
        contrib[i, :] = jnp.zeros((LANES,), jnp.float32)
        return 0
    lax.fori_loop(0, BLK, _zero_rows, 0)

    def _zero_acc(j, _):
        pltpu.sync_copy(contrib,
                        acc.at[pl.ds(s * ROWS_PER_TILE + j * BLK, BLK)])
        return 0
    lax.fori_loop(0, NZCOPY, _zero_acc, 0)
    pltpu.sync_copy(contrib.at[pl.ds(0, ZTAIL)],
                    acc.at[pl.ds(s * ROWS_PER_TILE + NZCOPY * BLK, ZTAIL)])
    plsc.subcore_barrier()

    svh = [svv[h] for h in range(HEADS)]
    dvh = [dvv[h] for h in range(HEADS)]
    iota = lax.iota(jnp.int32, LANES)
    cols_d = [jnp.full((LANES,), h, jnp.int32) for h in range(HEADS)]
    cols_n = [jnp.full((LANES,), HEADS + h, jnp.int32) for h in range(HEADS)]
    sem_ab = [sem_a, sem_b]

    nblk = BLK_PER + jnp.where(wid < BLK_REM, 1, 0)
    start = wid * BLK_PER + jnp.minimum(wid, BLK_REM)

    def _gather(k):
        sem = sem_ab[k % 2]
        cs = pltpu.async_copy(
            x_hbm.at[srcbuf.at[pl.ds(k * CHUNK, CHUNK)]],
            xsbuf.at[pl.ds(k * CHUNK, CHUNK)], sem)
        cd = pltpu.async_copy(
            x_hbm.at[dstbuf.at[k]],
            xdbuf.at[pl.ds(k * CHUNK, CHUNK)], sem)
        return cs, cd

    def _block(b, _):
        blk = start + b
        cp1 = pltpu.async_copy(src_hbm.at[blk], srcbuf, sem_g)
        cp2 = pltpu.async_copy(dst_hbm.at[blk], dstbuf, sem_g)
        cp1.wait()
        cp2.wait()
        pending = _gather(0)
        for k in range(NCHUNK):
            pending[0].wait()
            pending[1].wait()
            if k + 1 < NCHUNK:
                pending = _gather(k + 1)

            def _group(i, _):
                xs = xsbuf[pl.ds(k * CHUNK + i * LANES, LANES)]
                xd = xdbuf[pl.ds(k * CHUNK + i * LANES, LANES)]
                row_idx = iota + (k * CHUNK + i * LANES)
                for h in range(HEADS):
                    e = xs * svh[h] + xd * dvh[h]
                    e = jnp.maximum(e, e * jnp.float32(0.2))
                    ex = jnp.exp(e)
                    plsc.store_scatter(contrib, [row_idx, cols_d[h]], ex)
                    plsc.store_scatter(contrib, [row_idx, cols_n[h]], ex * xs)
                return 0

            lax.fori_loop(0, GPC, _group, 0)
            pltpu.async_copy(contrib.at[pl.ds(k * CHUNK, CHUNK)],
                             acc.at[dstbuf.at[k]], sem_s, add=True)
        pltpu.make_async_copy(contrib, acc.at[pl.ds(0, BLK)], sem_s).wait()
        return 0

    lax.fori_loop(0, nblk, _block, 0)
    plsc.subcore_barrier()

    pltpu.sync_copy(acc.at[pl.ds(s * ROWS_PER_TILE, ROWS_PER_TILE)],
                    part_hbm.at[c, pl.ds(s * ROWS_PER_TILE, ROWS_PER_TILE)])


@functools.partial(
    pl.kernel,
    mesh=plsc.VectorSubcoreMesh(core_axis_name="c", subcore_axis_name="s"),
    compiler_params=pltpu.CompilerParams(use_tc_tiling_on_sc=False,
                                         needs_layout_passes=False),
    out_type=jax.ShapeDtypeStruct((NC, N_NODES, 2 * HEADS), jnp.float32),
    scratch_types=[
        pltpu.VMEM((BLK,), jnp.int32),
        pltpu.VMEM((NCHUNK, CHUNK), jnp.int32),
        pltpu.VMEM((BLK,), jnp.float32),
        pltpu.VMEM((BLK,), jnp.float32),
        pltpu.VMEM((BLK, 2 * HEADS), jnp.float32),
        pltpu.VMEM((HEADS, LANES), jnp.float32),
        pltpu.VMEM((HEADS, LANES), jnp.float32),
        pltpu.VMEM_SHARED((N_NODES, 2 * HEADS), jnp.float32),
        pltpu.SemaphoreType.DMA,
        pltpu.SemaphoreType.DMA,
        pltpu.SemaphoreType.DMA,
        pltpu.SemaphoreType.DMA,
    ],
)
def _sc_edge_pass(x_hbm, src_hbm, dst_hbm, sv_hbm, dv_hbm, part_hbm,
                  srcbuf, dstbuf, xsbuf, xdbuf, contrib, svv, dvv,
                  acc, sem_g, sem_a, sem_b, sem_s):
    _sc_body(x_hbm, src_hbm, dst_hbm, sv_hbm, dv_hbm, part_hbm,
             srcbuf, dstbuf, xsbuf, xdbuf, contrib, svv, dvv,
             acc, sem_g, sem_a, sem_b, sem_s)


FIN_BN = 1000


def _finalize_body(p_ref, s_ref, b_ref, o_ref):
    p = p_ref[0] + p_ref[1]
    den = p[:, :HEADS]
    num = p[:, HEADS:]
    g = num / (den + jnp.float32(1e-16))
    o = jnp.dot(g, s_ref[...], preferred_element_type=jnp.float32)
    o_ref[...] = jnp.maximum(o + b_ref[...], jnp.float32(0.0))


_finalize = pl.pallas_call(
    _finalize_body,
    out_shape=jax.ShapeDtypeStruct((N_NODES, HEADS * OUT_CH), jnp.float32),
    grid=(N_NODES // FIN_BN,),
    in_specs=[
        pl.BlockSpec((NC, FIN_BN, 2 * HEADS), lambda i: (0, i, 0)),
        pl.BlockSpec((HEADS, HEADS * OUT_CH), lambda i: (0, 0)),
        pl.BlockSpec((1, HEADS * OUT_CH), lambda i: (0, 0)),
    ],
    out_specs=pl.BlockSpec((FIN_BN, HEADS * OUT_CH), lambda i: (i, 0)),
)


def kernel(x, edge_index, W, a_src, a_dst, bias):
    xf = x.reshape(N_NODES)
    src = edge_index[0].reshape(NBLOCKS, BLK)
    dst = edge_index[1].reshape(NBLOCKS, NCHUNK, CHUNK)
    Wr = W.reshape(HEADS, OUT_CH)
    s = jnp.sum(Wr * a_src, axis=1)
    d = jnp.sum(Wr * a_dst, axis=1)
    sv = jnp.broadcast_to(s[:, None], (HEADS, LANES))
    dv = jnp.broadcast_to(d[:, None], (HEADS, LANES))
    part = _sc_edge_pass(xf, src, dst, sv, dv)
    S = jnp.repeat(jnp.eye(HEADS, dtype=jnp.float32), OUT_CH, axis=1) * W
    out = _finalize(part, S, bias.reshape(1, HEADS * OUT_CH))
    return out


# speedup vs baseline: 1.2357x; 1.2357x over previous
import functools

import jax
import jax.numpy as jnp
from jax import lax
from jax.experimental import pallas as pl
from jax.experimental.pallas import tpu as pltpu
from jax.experimental.pallas import tpu_sc as plsc

N_NODES = 100000
N_EDGES = 1600000
HEADS = 8
OUT_CH = 8

NC = 2
NS = 16
NW = NC * NS
LANES = 16

BLK = 1280
CHUNK = 640
NCHUNK = BLK // CHUNK
GPC = CHUNK // LANES
NBLOCKS = N_EDGES // BLK
BLK_PER, BLK_REM = divmod(NBLOCKS, NW)
ROWS_PER_TILE = N_NODES // NS
NZCOPY = ROWS_PER_TILE // BLK
ZTAIL = ROWS_PER_TILE - NZCOPY * BLK


def _sc_body(x_hbm, src_hbm, dst_hbm, sv_hbm, dv_hbm, part_hbm,
             srcbuf, dstbuf, xsbuf, xdbuf, contrib, svv, dvv,
             acc, sem_g, sem_a, sem_s):
    c = lax.axis_index("c")
    s = lax.axis_index("s")
    wid = s * NC + c

    pltpu.sync_copy(sv_hbm, svv)
    pltpu.sync_copy(dv_hbm, dvv)

    def _zero_rows(i, _):
        contrib[i, :] = jnp.zeros((LANES,), jnp.float32)
        return 0
    lax.fori_loop(0, BLK, _zero_rows, 0)

    def _zero_acc(j, _):
        pltpu.sync_copy(contrib,
                        acc.at[pl.ds(s * ROWS_PER_TILE + j * BLK, BLK)])
        return 0
    lax.fori_loop(0, NZCOPY, _zero_acc, 0)
    pltpu.sync_copy(contrib.at[pl.ds(0, ZTAIL)],
                    acc.at[pl.ds(s * ROWS_PER_TILE + NZCOPY * BLK, ZTAIL)])
    plsc.subcore_barrier()

    svh = [svv[h] for h in range(HEADS)]
    dvh = [dvv[h] for h in range(HEADS)]
    iota = lax.iota(jnp.int32, LANES)
    cols_d = [jnp.full((LANES,), h, jnp.int32) for h in range(HEADS)]
    cols_n = [jnp.full((LANES,), HEADS + h, jnp.int32) for h in range(HEADS)]

    nblk = BLK_PER + jnp.where(wid < BLK_REM, 1, 0)
    start = wid * BLK_PER + jnp.minimum(wid, BLK_REM)

    def _block(b, _):
        blk = start + b
        cp1 = pltpu.async_copy(src_hbm.at[blk], srcbuf, sem_g)
        cp2 = pltpu.async_copy(dst_hbm.at[blk], dstbuf, sem_g)
        cp1.wait()
        cp2.wait()
        cps = []
        for k in range(NCHUNK):
            cps.append(pltpu.async_copy(
                x_hbm.at[srcbuf.at[pl.ds(k * CHUNK, CHUNK)]],
                xsbuf.at[pl.ds(k * CHUNK, CHUNK)], sem_a))
            cps.append(pltpu.async_copy(
                x_hbm.at[dstbuf.at[k]],
                xdbuf.at[pl.ds(k * CHUNK, CHUNK)], sem_a))
        for cp in cps:
            cp.wait()

        def _group(i, _):
            xs = xsbuf[pl.ds(i * LANES, LANES)]
            xd = xdbuf[pl.ds(i * LANES, LANES)]
            row_idx = iota + i * LANES
            for h in range(HEADS):
                e = xs * svh[h] + xd * dvh[h]
                e = jnp.maximum(e, e * jnp.float32(0.2))
                ex = jnp.exp(e)
                plsc.store_scatter(contrib, [row_idx, cols_d[h]], ex)
                plsc.store_scatter(contrib, [row_idx, cols_n[h]], ex * xs)
            return 0

        lax.fori_loop(0, BLK // LANES, _group, 0)

        for k in range(NCHUNK):
            pltpu.async_copy(contrib.at[pl.ds(k * CHUNK, CHUNK)],
                             acc.at[dstbuf.at[k]], sem_s, add=True)
        pltpu.make_async_copy(contrib, acc.at[pl.ds(0, BLK)], sem_s).wait()
        return 0

    lax.fori_loop(0, nblk, _block, 0)
    plsc.subcore_barrier()

    pltpu.sync_copy(acc.at[pl.ds(s * ROWS_PER_TILE, ROWS_PER_TILE)],
                    part_hbm.at[c, pl.ds(s * ROWS_PER_TILE, ROWS_PER_TILE)])


@functools.partial(
    pl.kernel,
    mesh=plsc.VectorSubcoreMesh(core_axis_name="c", subcore_axis_name="s"),
    compiler_params=pltpu.CompilerParams(use_tc_tiling_on_sc=False,
                                         needs_layout_passes=False),
    out_type=jax.ShapeDtypeStruct((NC, N_NODES, 2 * HEADS), jnp.float32),
    scratch_types=[
        pltpu.VMEM((BLK,), jnp.int32),
        pltpu.VMEM((NCHUNK, CHUNK), jnp.int32),
        pltpu.VMEM((BLK,), jnp.float32),
        pltpu.VMEM((BLK,), jnp.float32),
        pltpu.VMEM((BLK, 2 * HEADS), jnp.float32),
        pltpu.VMEM((HEADS, LANES), jnp.float32),
        pltpu.VMEM((HEADS, LANES), jnp.float32),
        pltpu.VMEM_SHARED((N_NODES, 2 * HEADS), jnp.float32),
        pltpu.SemaphoreType.DMA,
        pltpu.SemaphoreType.DMA,
        pltpu.SemaphoreType.DMA,
    ],
)
def _sc_edge_pass(x_hbm, src_hbm, dst_hbm, sv_hbm, dv_hbm, part_hbm,
                  srcbuf, dstbuf, xsbuf, xdbuf, contrib, svv, dvv,
                  acc, sem_g, sem_a, sem_s):
    _sc_body(x_hbm, src_hbm, dst_hbm, sv_hbm, dv_hbm, part_hbm,
             srcbuf, dstbuf, xsbuf, xdbuf, contrib, svv, dvv,
             acc, sem_g, sem_a, sem_s)


FIN_BN = 1000


def _finalize_body(p_ref, s_ref, b_ref, o_ref):
    p = p_ref[0] + p_ref[1]
    den = p[:, :HEADS]
    num = p[:, HEADS:]
    g = num / (den + jnp.float32(1e-16))
    o = jnp.dot(g, s_ref[...], preferred_element_type=jnp.float32)
    o_ref[...] = jnp.maximum(o + b_ref[...], jnp.float32(0.0))


_finalize = pl.pallas_call(
    _finalize_body,
    out_shape=jax.ShapeDtypeStruct((N_NODES, HEADS * OUT_CH), jnp.float32),
    grid=(N_NODES // FIN_BN,),
    in_specs=[
        pl.BlockSpec((NC, FIN_BN, 2 * HEADS), lambda i: (0, i, 0)),
        pl.BlockSpec((HEADS, HEADS * OUT_CH), lambda i: (0, 0)),
        pl.BlockSpec((1, HEADS * OUT_CH), lambda i: (0, 0)),
    ],
    out_specs=pl.BlockSpec((FIN_BN, HEADS * OUT_CH), lambda i: (i, 0)),
)


def kernel(x, edge_index, W, a_src, a_dst, bias):
    xf = x.reshape(N_NODES)
    src = edge_index[0].reshape(NBLOCKS, BLK)
    dst = edge_index[1].reshape(NBLOCKS, NCHUNK, CHUNK)
    Wr = W.reshape(HEADS, OUT_CH)
    s = jnp.sum(Wr * a_src, axis=1)
    d = jnp.sum(Wr * a_dst, axis=1)
    sv = jnp.broadcast_to(s[:, None], (HEADS, LANES))
    dv = jnp.broadcast_to(d[:, None], (HEADS, LANES))
    part = _sc_edge_pass(xf, src, dst, sv, dv)
    S = jnp.repeat(jnp.eye(HEADS, dtype=jnp.float32), OUT_CH, axis=1) * W
    out = _finalize(part, S, bias.reshape(1, HEADS * OUT_CH))
    return out


# speedup vs baseline: 1.3038x; 1.0551x over previous
import functools

import jax
import jax.numpy as jnp
from jax import lax
from jax.experimental import pallas as pl
from jax.experimental.pallas import tpu as pltpu
from jax.experimental.pallas import tpu_sc as plsc

N_NODES = 100000
N_EDGES = 1600000
HEADS = 8
OUT_CH = 8

NC = 2
NS = 16
NW = NC * NS
LANES = 16

BLK = 640
CHUNK = 640
NCHUNK = BLK // CHUNK
GPC = CHUNK // LANES
NBLOCKS = N_EDGES // BLK
BLK_PER, BLK_REM = divmod(NBLOCKS, NW)
ROWS_PER_TILE = N_NODES // NS
NZCOPY = ROWS_PER_TILE // BLK
ZTAIL = ROWS_PER_TILE - NZCOPY * BLK
XST = 6248


def _sc_body(x_hbm, src_hbm, dst_hbm, sv_hbm, dv_hbm, part_hbm,
             srcbuf, dstbuf, xsbuf, xdbuf, contrib, svv, dvv,
             acc, x_sh, sem_g, sem_a, sem_s):
    c = lax.axis_index("c")
    s = lax.axis_index("s")
    wid = s * NC + c

    pltpu.sync_copy(sv_hbm, svv)
    pltpu.sync_copy(dv_hbm, dvv)

    pltpu.sync_copy(x_hbm.at[pl.ds(s * XST, XST)], x_sh.at[pl.ds(s * XST, XST)])

    @pl.when(s == NS - 1)
    def _():
        pltpu.sync_copy(x_hbm.at[pl.ds(NS * XST, N_NODES - NS * XST)],
                        x_sh.at[pl.ds(NS * XST, N_NODES - NS * XST)])

    def _zero_rows(i, _):
        contrib[i, :] = jnp.zeros((LANES,), jnp.float32)
        return 0
    lax.fori_loop(0, BLK, _zero_rows, 0)

    def _zero_acc(j, _):
        pltpu.sync_copy(contrib,
                        acc.at[pl.ds(s * ROWS_PER_TILE + j * BLK, BLK)])
        return 0
    lax.fori_loop(0, NZCOPY, _zero_acc, 0)
    pltpu.sync_copy(contrib.at[pl.ds(0, ZTAIL)],
                    acc.at[pl.ds(s * ROWS_PER_TILE + NZCOPY * BLK, ZTAIL)])
    plsc.subcore_barrier()

    svh = [svv[h] for h in range(HEADS)]
    dvh = [dvv[h] for h in range(HEADS)]
    iota = lax.iota(jnp.int32, LANES)
    cols_d = [jnp.full((LANES,), h, jnp.int32) for h in range(HEADS)]
    cols_n = [jnp.full((LANES,), HEADS + h, jnp.int32) for h in range(HEADS)]

    nblk = BLK_PER + jnp.where(wid < BLK_REM, 1, 0)
    start = wid * BLK_PER + jnp.minimum(wid, BLK_REM)

    def _block(b, _):
        blk = start + b
        cp1 = pltpu.async_copy(src_hbm.at[blk], srcbuf, sem_g)
        cp2 = pltpu.async_copy(dst_hbm.at[blk], dstbuf, sem_g)
        cp1.wait()
        cp2.wait()
        cps = []
        for k in range(NCHUNK):
            cps.append(pltpu.async_copy(
                x_sh.at[srcbuf.at[pl.ds(k * CHUNK, CHUNK)]],
                xsbuf.at[pl.ds(k * CHUNK, CHUNK)], sem_a))
            cps.append(pltpu.async_copy(
                x_sh.at[dstbuf.at[k]],
                xdbuf.at[pl.ds(k * CHUNK, CHUNK)], sem_a))
        for cp in cps:
            cp.wait()

        def _group(i, _):
            xs = xsbuf[pl.ds(i * LANES, LANES)]
            xd = xdbuf[pl.ds(i * LANES, LANES)]
            row_idx = iota + i * LANES
            for h in range(HEADS):
                e = xs * svh[h] + xd * dvh[h]
                e = jnp.maximum(e, e * jnp.float32(0.2))
                ex = jnp.exp(e)
                plsc.store_scatter(contrib, [row_idx, cols_d[h]], ex)
                plsc.store_scatter(contrib, [row_idx, cols_n[h]], ex * xs)
            return 0

        lax.fori_loop(0, BLK // LANES, _group, 0)

        for k in range(NCHUNK):
            pltpu.async_copy(contrib.at[pl.ds(k * CHUNK, CHUNK)],
                             acc.at[dstbuf.at[k]], sem_s, add=True)
        pltpu.make_async_copy(contrib, acc.at[pl.ds(0, BLK)], sem_s).wait()
        return 0

    lax.fori_loop(0, nblk, _block, 0)
    plsc.subcore_barrier()

    pltpu.sync_copy(acc.at[pl.ds(s * ROWS_PER_TILE, ROWS_PER_TILE)],
                    part_hbm.at[c, pl.ds(s * ROWS_PER_TILE, ROWS_PER_TILE)])


@functools.partial(
    pl.kernel,
    mesh=plsc.VectorSubcoreMesh(core_axis_name="c", subcore_axis_name="s"),
    compiler_params=pltpu.CompilerParams(use_tc_tiling_on_sc=False,
                                         needs_layout_passes=False),
    out_type=jax.ShapeDtypeStruct((NC, N_NODES, 2 * HEADS), jnp.float32),
    scratch_types=[
        pltpu.VMEM((BLK,), jnp.int32),
        pltpu.VMEM((NCHUNK, CHUNK), jnp.int32),
        pltpu.VMEM((BLK,), jnp.float32),
        pltpu.VMEM((BLK,), jnp.float32),
        pltpu.VMEM((BLK, 2 * HEADS), jnp.float32),
        pltpu.VMEM((HEADS, LANES), jnp.float32),
        pltpu.VMEM((HEADS, LANES), jnp.float32),
        pltpu.VMEM_SHARED((N_NODES, 2 * HEADS), jnp.float32),
        pltpu.VMEM_SHARED((N_NODES,), jnp.float32),
        pltpu.SemaphoreType.DMA,
        pltpu.SemaphoreType.DMA,
        pltpu.SemaphoreType.DMA,
    ],
)
def _sc_edge_pass(x_hbm, src_hbm, dst_hbm, sv_hbm, dv_hbm, part_hbm,
                  srcbuf, dstbuf, xsbuf, xdbuf, contrib, svv, dvv,
                  acc, x_sh, sem_g, sem_a, sem_s):
    _sc_body(x_hbm, src_hbm, dst_hbm, sv_hbm, dv_hbm, part_hbm,
             srcbuf, dstbuf, xsbuf, xdbuf, contrib, svv, dvv,
             acc, x_sh, sem_g, sem_a, sem_s)


FIN_BN = 1000


def _finalize_body(p_ref, s_ref, b_ref, o_ref):
    p = p_ref[0] + p_ref[1]
    den = p[:, :HEADS]
    num = p[:, HEADS:]
    g = num / (den + jnp.float32(1e-16))
    o = jnp.dot(g, s_ref[...], preferred_element_type=jnp.float32)
    o_ref[...] = jnp.maximum(o + b_ref[...], jnp.float32(0.0))


_finalize = pl.pallas_call(
    _finalize_body,
    out_shape=jax.ShapeDtypeStruct((N_NODES, HEADS * OUT_CH), jnp.float32),
    grid=(N_NODES // FIN_BN,),
    in_specs=[
        pl.BlockSpec((NC, FIN_BN, 2 * HEADS), lambda i: (0, i, 0)),
        pl.BlockSpec((HEADS, HEADS * OUT_CH), lambda i: (0, 0)),
        pl.BlockSpec((1, HEADS * OUT_CH), lambda i: (0, 0)),
    ],
    out_specs=pl.BlockSpec((FIN_BN, HEADS * OUT_CH), lambda i: (i, 0)),
)


def kernel(x, edge_index, W, a_src, a_dst, bias):
    xf = x.reshape(N_NODES)
    src = edge_index[0].reshape(NBLOCKS, BLK)
    dst = edge_index[1].reshape(NBLOCKS, NCHUNK, CHUNK)
    Wr = W.reshape(HEADS, OUT_CH)
    s = jnp.sum(Wr * a_src, axis=1)
    d = jnp.sum(Wr * a_dst, axis=1)
    sv = jnp.broadcast_to(s[:, None], (HEADS, LANES))
    dv = jnp.broadcast_to(d[:, None], (HEADS, LANES))
    part = _sc_edge_pass(xf, src, dst, sv, dv)
    S = jnp.repeat(jnp.eye(HEADS, dtype=jnp.float32), OUT_CH, axis=1) * W
    out = _finalize(part, S, bias.reshape(1, HEADS * OUT_CH))
    return out
